# baseline (speedup 1.0000x reference)
import jax
import jax.numpy as jnp
from jax import lax
from jax.experimental import pallas as pl
from jax.experimental.pallas import tpu as pltpu
from jax.experimental.pallas import tpu_sc as plsc

DIM = 1024
E = 8
TOPK = 2
INTER = 512
T = 2048
A = T * TOPK
BLK = 128
NB = A // BLK + E
P = NB * BLK
BT = 256
NC, NS = 2, 16
NW = NC * NS
TPW = T // NW
SPW = P // NW
GCH = SPW // 2
CCH = 32


def _silu(v):
    return v * jax.nn.sigmoid(v)


def _mmT(a, b):
    return jax.lax.dot_general(a, b, (((1,), (1,)), ((), ())),
                               preferred_element_type=jnp.float32)


def _route_body(x_ref, wg_ref, bias_ref,
                d0_ref, d1_ref, w0_ref, w1_ref, be_ref):
    x = x_ref[...]
    scores = jax.nn.sigmoid(_mmT(x, wg_ref[...]))
    biased = scores + bias_ref[...]
    lane = jax.lax.broadcasted_iota(jnp.int32, (T, E), 1)
    m0 = jnp.max(biased, axis=1, keepdims=True)
    i0 = jnp.min(jnp.where(biased == m0, lane, E), axis=1, keepdims=True)
    masked = jnp.where(lane == i0, -jnp.inf, biased)
    m1 = jnp.max(masked, axis=1, keepdims=True)
    i1 = jnp.min(jnp.where(masked == m1, lane, E), axis=1, keepdims=True)
    w0_ref[...] = jnp.sum(jnp.where(lane == i0, scores, 0.0), axis=1,
                          keepdims=True)
    w1_ref[...] = jnp.sum(jnp.where(lane == i1, scores, 0.0), axis=1,
                          keepdims=True)
    cnt = ((lane == i0).astype(jnp.float32)
           + (lane == i1).astype(jnp.float32))
    r2 = jax.lax.broadcasted_iota(jnp.int32, (T, T), 0)
    c2 = jax.lax.broadcasted_iota(jnp.int32, (T, T), 1)
    tri = (c2 <= r2).astype(jnp.bfloat16)
    inc = jax.lax.dot_general(tri, cnt.astype(jnp.bfloat16),
                              (((1,), (0,)), ((), ())),
                              preferred_element_type=jnp.float32)
    exc = inc - cnt
    counts = inc[T - 1:T, :]
    nb = jnp.floor((counts + (BLK - 1)) * (1.0 / BLK))
    r8 = jax.lax.broadcasted_iota(jnp.int32, (E, E), 0)
    c8 = jax.lax.broadcasted_iota(jnp.int32, (E, E), 1)
    su = (r8 < c8).astype(jnp.float32)
    offb = jax.lax.dot_general(nb, su, (((1,), (0,)), ((), ())),
                               preferred_element_type=jnp.float32)
    offs = offb * float(BLK)
    d0 = jnp.sum(jnp.where(lane == i0, exc + offs, 0.0), axis=1, keepdims=True)
    d1 = jnp.sum(jnp.where(lane == i1, exc + offs, 0.0), axis=1, keepdims=True)
    d0_ref[...] = d0.astype(jnp.int32)
    d1_ref[...] = d1.astype(jnp.int32)
    offb_col = jnp.sum(jnp.where(r8 == c8, jnp.broadcast_to(offb, (E, E)),
                                 0.0), axis=1, keepdims=True)
    jb = jax.lax.broadcasted_iota(jnp.int32, (E, NB), 1).astype(jnp.float32)
    be = jnp.sum((jb >= offb_col).astype(jnp.float32), axis=0,
                 keepdims=True) - 1.0
    be_ref[...] = be.astype(jnp.int32)


def _dispatch_body(d0_hbm, d1_hbm, w0_hbm, w1_hbm, tok_hbm, zi_hbm, zf_hbm,
                   x_hbm, xs_hbm, ws_hbm,
                   stok, sw, d0_v, d1_v, tk_v, w0_v, w1_v, wsl_v,
                   idx_v, rows_v, sem):
    c = lax.axis_index("c")
    s = lax.axis_index("s")
    @pl.when(s == 0)
    def _():
        pltpu.sync_copy(zi_hbm, stok)
        pltpu.sync_copy(zf_hbm, sw)
    plsc.subcore_barrier()
    tbase = s * (T // NS)
    pltpu.sync_copy(d0_hbm.at[pl.ds(tbase, T // NS)], d0_v)
    pltpu.sync_copy(d1_hbm.at[pl.ds(tbase, T // NS)], d1_v)
    pltpu.sync_copy(tok_hbm.at[pl.ds(tbase, T // NS)], tk_v)
    pltpu.sync_copy(w0_hbm.at[pl.ds(tbase, T // NS)], w0_v)
    pltpu.sync_copy(w1_hbm.at[pl.ds(tbase, T // NS)], w1_v)
    pltpu.sync_copy(tk_v, stok.at[d0_v], add=True)
    pltpu.sync_copy(tk_v, stok.at[d1_v], add=True)
    pltpu.sync_copy(w0_v, sw.at[d0_v], add=True)
    pltpu.sync_copy(w1_v, sw.at[d1_v], add=True)
    plsc.subcore_barrier()
    slot0 = c * (P // NC) + s * SPW
    pltpu.sync_copy(sw.at[pl.ds(slot0, SPW)], wsl_v)
    pltpu.sync_copy(wsl_v, ws_hbm.at[pl.ds(slot0, SPW)])
    pltpu.sync_copy(stok.at[pl.ds(slot0, SPW)], idx_v)
    for k in range(SPW // GCH):
        pltpu.async_copy(x_hbm.at[idx_v.at[pl.ds(k * GCH, GCH)]], rows_v,
                         sem).wait()
        pltpu.sync_copy(rows_v, xs_hbm.at[pl.ds(slot0 + k * GCH, GCH)])


def _grouped_body(be_ref, xs_ref, w1_ref, w3_ref, w2_ref, wsl_ref, eo_ref):
    x = xs_ref[...]
    h = _silu(_mmT(x, w1_ref[0])) * _mmT(x, w3_ref[0])
    eo = _mmT(h, w2_ref[0])
    eo_ref[...] = eo * wsl_ref[0]


def _combine_body(d0_hbm, d1_hbm, eos_hbm, y0_hbm, y1_hbm,
                  idx_v, rows_v, sem):
    wid = lax.axis_index("c") * NS + lax.axis_index("s")
    base = wid * TPW
    for k in range(TPW // CCH):
        pltpu.sync_copy(d0_hbm.at[pl.ds(base + k * CCH, CCH)], idx_v)
        pltpu.async_copy(eos_hbm.at[idx_v], rows_v, sem).wait()
        pltpu.sync_copy(rows_v, y0_hbm.at[pl.ds(base + k * CCH, CCH)])
    for k in range(TPW // CCH):
        pltpu.sync_copy(d1_hbm.at[pl.ds(base + k * CCH, CCH)], idx_v)
        pltpu.async_copy(eos_hbm.at[idx_v], rows_v, sem).wait()
        pltpu.sync_copy(rows_v, y1_hbm.at[pl.ds(base + k * CCH, CCH)])


def _shared_body(x_ref, y0_ref, y1_ref, ws1_ref, ws3_ref, ws2_ref, o_ref):
    x = x_ref[...]
    h = _silu(_mmT(x, ws1_ref[...])) * _mmT(x, ws3_ref[...])
    z = _mmT(h, ws2_ref[...])
    o_ref[...] = z + y0_ref[...] + y1_ref[...]


@jax.jit
def _run(x, Wg, expert_bias, W1, W2, W3, Ws1, Ws2, Ws3):
    shape = x.shape
    xf = x.reshape(-1, DIM)
    bias2 = expert_bias.reshape(1, E)
    f32 = jnp.float32
    i32 = jnp.int32

    d0, d1, w0, w1, be = pl.pallas_call(
        _route_body,
        out_shape=(
            jax.ShapeDtypeStruct((T, 1), i32),
            jax.ShapeDtypeStruct((T, 1), i32),
            jax.ShapeDtypeStruct((T, 1), f32),
            jax.ShapeDtypeStruct((T, 1), f32),
            jax.ShapeDtypeStruct((1, NB), i32),
        ),
    )(xf, Wg, bias2)
    d0 = d0.reshape(T)
    d1 = d1.reshape(T)

    tok = lax.iota(i32, T)
    zi = jnp.zeros((P,), i32)
    zf = jnp.zeros((P,), f32)
    mesh = plsc.VectorSubcoreMesh(core_axis_name="c", subcore_axis_name="s",
                                  num_cores=NC, num_subcores=NS)
    xs, ws = pl.kernel(
        _dispatch_body,
        out_type=(
            jax.ShapeDtypeStruct((P, DIM), f32),
            jax.ShapeDtypeStruct((P,), f32),
        ),
        mesh=mesh,
        scratch_types=[
            pltpu.VMEM_SHARED((P,), i32),
            pltpu.VMEM_SHARED((P,), f32),
            pltpu.VMEM((T // NS,), i32),
            pltpu.VMEM((T // NS,), i32),
            pltpu.VMEM((T // NS,), i32),
            pltpu.VMEM((T // NS,), f32),
            pltpu.VMEM((T // NS,), f32),
            pltpu.VMEM((SPW,), f32),
            pltpu.VMEM((SPW,), i32),
            pltpu.VMEM((GCH, DIM), f32),
            pltpu.SemaphoreType.DMA,
        ],
    )(d0, d1, w0.reshape(T), w1.reshape(T), tok, zi, zf, xf)

    eos = pl.pallas_call(
        _grouped_body,
        grid_spec=pltpu.PrefetchScalarGridSpec(
            num_scalar_prefetch=1,
            grid=(NB,),
            in_specs=[
                pl.BlockSpec((BLK, DIM), lambda i, be: (i, 0)),
                pl.BlockSpec((1, INTER, DIM), lambda i, be: (be[i], 0, 0)),
                pl.BlockSpec((1, INTER, DIM), lambda i, be: (be[i], 0, 0)),
                pl.BlockSpec((1, DIM, INTER), lambda i, be: (be[i], 0, 0)),
                pl.BlockSpec((1, BLK, 1), lambda i, be: (i, 0, 0)),
            ],
            out_specs=pl.BlockSpec((BLK, DIM), lambda i, be: (i, 0)),
        ),
        out_shape=jax.ShapeDtypeStruct((P, DIM), f32),
    )(be.reshape(NB), xs, W1, W3, W2, ws.reshape(NB, BLK, 1))

    y0, y1 = pl.kernel(
        _combine_body,
        out_type=(
            jax.ShapeDtypeStruct((T, DIM), f32),
            jax.ShapeDtypeStruct((T, DIM), f32),
        ),
        mesh=mesh,
        scratch_types=[
            pltpu.VMEM((CCH,), i32),
            pltpu.VMEM((CCH, DIM), f32),
            pltpu.SemaphoreType.DMA,
        ],
    )(d0, d1, eos)

    out = pl.pallas_call(
        _shared_body,
        grid=(T // BT,),
        in_specs=[
            pl.BlockSpec((BT, DIM), lambda i: (i, 0)),
            pl.BlockSpec((BT, DIM), lambda i: (i, 0)),
            pl.BlockSpec((BT, DIM), lambda i: (i, 0)),
            pl.BlockSpec((2 * INTER, DIM), lambda i: (0, 0)),
            pl.BlockSpec((2 * INTER, DIM), lambda i: (0, 0)),
            pl.BlockSpec((DIM, 2 * INTER), lambda i: (0, 0)),
        ],
        out_specs=pl.BlockSpec((BT, DIM), lambda i: (i, 0)),
        out_shape=jax.ShapeDtypeStruct((T, DIM), f32),
    )(xf, y0, y1, Ws1, Ws3, Ws2)

    return out.reshape(shape)


def kernel(x, Wg, expert_bias, W1, W2, W3, Ws1, Ws2, Ws3):
    return _run(x, Wg, expert_bias, W1, W2, W3, Ws1, Ws2, Ws3)
